# Initial kernel scaffold; baseline (speedup 1.0000x reference)
import jax
import jax.numpy as jnp
from jax.experimental import pallas as pl


def kernel(x, centroids, w1, w3, w2, bias, scale):
    raise NotImplementedError("write your pallas kernel here")


# speedup vs baseline: 3.0421x; 3.0421x over previous
import functools

import jax
import jax.numpy as jnp
from jax.experimental import pallas as pl
from jax.experimental.pallas import tpu as pltpu

TOKENS = 2048
D_MODEL = 768
D_EXPERT = 384
E = 8
TOP_K = 2


def _router_body(x_ref, c_ref, b_ref, combine_ref, fi_ref, pi_ref, bal_ref,
                 dl_ref, bm_ref, bs_ref):
    x = x_ref[...]
    c = c_ref[...]
    b = b_ref[...]
    logits = jax.lax.dot_general(
        x, c, (((1,), (1,)), ((), ())),
        preferred_element_type=jnp.float32) + b
    scores = jax.nn.sigmoid(logits)
    e_iota = jax.lax.broadcasted_iota(jnp.int32, scores.shape, 1)
    m1 = jnp.max(scores, axis=1, keepdims=True)
    idx1 = jnp.min(jnp.where(scores == m1, e_iota, E), axis=1, keepdims=True)
    oh1 = (e_iota == idx1).astype(jnp.float32)
    masked = jnp.where(e_iota == idx1, -jnp.inf, scores)
    m2 = jnp.max(masked, axis=1, keepdims=True)
    idx2 = jnp.min(jnp.where(masked == m2, e_iota, E), axis=1, keepdims=True)
    oh2 = (e_iota == idx2).astype(jnp.float32)
    denom = jnp.clip(m1 + m2, 1e-9, None)
    g1 = m1 / denom
    g2 = m2 / denom
    combine = g1 * oh1 + g2 * oh2
    combine_ref[...] = combine
    sel = oh1 + oh2
    counts = jnp.sum(sel, axis=0, keepdims=True)
    n = x.shape[0]
    fi = counts / (n * TOP_K)
    pi = jnp.sum(combine, axis=0, keepdims=True) / n
    fi_ref[...] = fi
    pi_ref[...] = pi
    bal_ref[...] = jnp.sum(fi * pi, keepdims=True).reshape(1, 1)
    dl_ref[...] = jnp.sum(fi, keepdims=True).reshape(1, 1)
    bm = jnp.mean(b)
    bm_ref[...] = bm.reshape(1, 1)
    bs_ref[...] = jnp.sqrt(jnp.sum((b - bm) ** 2) / (E - 1)).reshape(1, 1)


def _expert_body(x_ref, w1_ref, w3_ref, w2_ref, combine_ref, scale_ref, out_ref):
    e = pl.program_id(0)
    x = x_ref[...]
    w1 = w1_ref[0]
    w3 = w3_ref[0]
    w2 = w2_ref[0]
    h1 = jnp.dot(x, w1, preferred_element_type=jnp.float32)
    h3 = jnp.dot(x, w3, preferred_element_type=jnp.float32)
    h = h1 * jax.nn.sigmoid(h1) * h3
    eo = jnp.dot(h, w2, preferred_element_type=jnp.float32)
    combine = combine_ref[...]
    e_iota = jax.lax.broadcasted_iota(jnp.int32, combine.shape, 1)
    col = jnp.sum(jnp.where(e_iota == e, combine, 0.0), axis=1, keepdims=True)
    contrib = eo * col * scale_ref[0, 0]

    @pl.when(e == 0)
    def _():
        out_ref[...] = contrib

    @pl.when(e > 0)
    def _():
        out_ref[...] += contrib


@jax.jit
def kernel(x, centroids, w1, w3, w2, bias, scale):
    n = x.shape[0]
    b2 = bias.reshape(1, E)
    combine, fi, pi, bal, dl, bm, bs = pl.pallas_call(
        _router_body,
        out_shape=(
            jax.ShapeDtypeStruct((n, E), jnp.float32),
            jax.ShapeDtypeStruct((1, E), jnp.float32),
            jax.ShapeDtypeStruct((1, E), jnp.float32),
            jax.ShapeDtypeStruct((1, 1), jnp.float32),
            jax.ShapeDtypeStruct((1, 1), jnp.float32),
            jax.ShapeDtypeStruct((1, 1), jnp.float32),
            jax.ShapeDtypeStruct((1, 1), jnp.float32),
        ),
    )(x, centroids, b2)

    out = pl.pallas_call(
        _expert_body,
        grid=(E,),
        in_specs=[
            pl.BlockSpec((n, D_MODEL), lambda e: (0, 0)),
            pl.BlockSpec((1, D_MODEL, D_EXPERT), lambda e: (e, 0, 0)),
            pl.BlockSpec((1, D_MODEL, D_EXPERT), lambda e: (e, 0, 0)),
            pl.BlockSpec((1, D_EXPERT, D_MODEL), lambda e: (e, 0, 0)),
            pl.BlockSpec((n, E), lambda e: (0, 0)),
            pl.BlockSpec((1, 1), lambda e: (0, 0)),
        ],
        out_specs=pl.BlockSpec((n, D_MODEL), lambda e: (0, 0)),
        out_shape=jax.ShapeDtypeStruct((n, D_MODEL), jnp.float32),
        compiler_params=pltpu.CompilerParams(
            dimension_semantics=("arbitrary",),
        ),
    )(x, w1, w3, w2, combine, scale.reshape(1, 1))

    return (out, fi.reshape(E), pi.reshape(E), bal.reshape(()),
            dl.reshape(1), bm.reshape(()), bs.reshape(()))
